# Initial kernel scaffold; baseline (speedup 1.0000x reference)
import jax
import jax.numpy as jnp
from jax.experimental import pallas as pl


def kernel(_x, rand_index):
    raise NotImplementedError("write your pallas kernel here")


# speedup vs baseline: 16.5553x; 16.5553x over previous
import jax
import jax.numpy as jnp
from jax.experimental import pallas as pl
from jax.experimental.pallas import tpu as pltpu

N_GRAM_W = 4
BR = 512


def _fused_body(x_ref, mask_ref, o_ref, carry_ref):
    i = pl.program_id(0)
    xblk = x_ref[...]
    carry = carry_ref[...]
    seq_blocks = mask_ref.shape[0]
    del seq_blocks
    carry = jnp.where((i * BR) % S_STATIC == 0, jnp.zeros_like(carry), carry)
    ext = jnp.concatenate([carry[8 - (N_GRAM_W - 1):], xblk], axis=0)
    m = ext[N_GRAM_W - 1:]
    for g in range(1, N_GRAM_W):
        m = jnp.minimum(m, ext[N_GRAM_W - 1 - g: N_GRAM_W - 1 - g + BR])
    sig = jax.nn.sigmoid(xblk)
    keep = mask_ref[...] > 0.0
    o_ref[...] = jnp.where(keep, sig * m + (1.0 - sig) * xblk, xblk)
    new_carry = carry_ref[...]
    carry_ref[...] = jnp.concatenate(
        [new_carry[:8 - (N_GRAM_W - 1)], xblk[BR - (N_GRAM_W - 1):]], axis=0)


S_STATIC = 8192


def kernel(_x, rand_index):
    B, S, H = _x.shape
    assert S == S_STATIC and S % BR == 0
    T = B * S
    xf = _x.reshape(T, H)
    mask = jnp.zeros((T, 1), jnp.float32).at[rand_index].set(1.0)

    out = pl.pallas_call(
        _fused_body,
        grid=(T // BR,),
        in_specs=[
            pl.BlockSpec((BR, H), lambda i: (i, 0)),
            pl.BlockSpec((BR, 1), lambda i: (i, 0)),
        ],
        out_specs=pl.BlockSpec((BR, H), lambda i: (i, 0)),
        out_shape=jax.ShapeDtypeStruct((T, H), jnp.float32),
        scratch_shapes=[pltpu.VMEM((8, H), jnp.float32)],
    )(xf, mask)
    return out.reshape(B, S, H)


# speedup vs baseline: 17.4597x; 1.0546x over previous
import jax
import jax.numpy as jnp
from jax.experimental import pallas as pl
from jax.experimental.pallas import tpu as pltpu

BR = 512
S_STATIC = 8192


def _fused_body(x_ref, mask_ref, o_ref, carry_ref):
    i = pl.program_id(0)
    xblk = x_ref[...]
    carry = carry_ref[...]
    carry = jnp.where((i * BR) % S_STATIC == 0, jnp.zeros_like(carry), carry)
    ext = jnp.concatenate([carry[5:], xblk], axis=0)
    m2 = jnp.minimum(ext[1:], ext[:-1])
    m = jnp.minimum(m2[2:], m2[:BR])
    sig = jax.nn.sigmoid(xblk)
    w = mask_ref[...] * sig
    o_ref[...] = xblk + w * (m - xblk)
    carry_ref[...] = xblk[BR - 8:]


def kernel(_x, rand_index):
    B, S, H = _x.shape
    assert S == S_STATIC and S % BR == 0
    T = B * S
    xf = _x.reshape(T, H)
    mask = jnp.zeros((T, 1), jnp.float32).at[rand_index].set(1.0)

    out = pl.pallas_call(
        _fused_body,
        grid=(T // BR,),
        in_specs=[
            pl.BlockSpec((BR, H), lambda i: (i, 0)),
            pl.BlockSpec((BR, 1), lambda i: (i, 0)),
        ],
        out_specs=pl.BlockSpec((BR, H), lambda i: (i, 0)),
        out_shape=jax.ShapeDtypeStruct((T, H), jnp.float32),
        scratch_shapes=[pltpu.VMEM((8, H), jnp.float32)],
    )(xf, mask)
    return out.reshape(B, S, H)
